# baseline (speedup 1.0000x reference)
import jax
import jax.numpy as jnp
from jax.experimental import pallas as pl
from jax.experimental.pallas import tpu as pltpu


def _make_fused_kernel(H, W, C, TS):
    TWO_C = 2 * C
    THREE_C = 3 * C
    SIX_C = 6 * C
    T = H // TS

    def _body(xc_ref, xt_ref, xb_ref, w_ref, b_ref, o_ref, xp_ref):
        ti = pl.program_id(1)

        xt = jnp.transpose(xc_ref[0].astype(jnp.bfloat16))

        zrow = jnp.zeros((1, W, C), jnp.bfloat16)
        zcol = jnp.zeros((TS + 2, 1, C), jnp.bfloat16)

        xp_ref[:, 0:1, :] = zcol
        xp_ref[:, W + 1:W + 2, :] = zcol
        xp_ref[1:TS + 1, 1:W + 1, :] = xt.reshape(TS, W, C)

        @pl.when(ti == 0)
        def _():
            xp_ref[0:1, 1:W + 1, :] = zrow

        @pl.when(ti > 0)
        def _():
            xp_ref[0:1, 1:W + 1, :] = (
                jnp.transpose(xt_ref[0, :, 128 - W:].astype(jnp.bfloat16))
                .reshape(1, W, C))

        @pl.when(ti == T - 1)
        def _():
            xp_ref[TS + 1:TS + 2, 1:W + 1, :] = zrow

        @pl.when(ti < T - 1)
        def _():
            xp_ref[TS + 1:TS + 2, 1:W + 1, :] = (
                jnp.transpose(xb_ref[0, :, :W].astype(jnp.bfloat16))
                .reshape(1, W, C))

        xp = xp_ref[...]
        taps = []
        for r in range(3):
            for s in range(3):
                taps.append(xp[r:r + TS, s:s + W, :].reshape(TS * W, C))
        patches = jnp.concatenate(taps, axis=-1)

        acc0 = jnp.dot(patches[:, :SIX_C], w_ref[0],
                       preferred_element_type=jnp.float32) + b_ref[0]
        acc1 = jnp.dot(patches[:, THREE_C:], w_ref[1],
                       preferred_element_type=jnp.float32) + b_ref[1]

        for di, acc in ((0, acc0), (1, acc1)):
            a0 = jnp.transpose(acc[:, :C])
            a1 = jnp.transpose(acc[:, C:])
            ilv = jnp.stack([a0, a1], axis=-1).reshape(C, 2 * TS * W)
            for i in range(TS):
                p = 2 * i + di
                o_ref[0, :, p * 2 * W:(p + 1) * 2 * W] = (
                    ilv[:, i * 2 * W:(i + 1) * 2 * W])

    return _body


def kernel(x, w_cat, b_cat):
    n, c, h, w = x.shape
    x3 = x.reshape(n, c, h * w)

    wr = w_cat.reshape(2, 3, 3 * c, 2 * c)
    w6 = jnp.stack([wr[0, 0:2].reshape(6 * c, 2 * c),
                    wr[1, 1:3].reshape(6 * c, 2 * c)]).astype(jnp.bfloat16)
    b2 = b_cat.astype(jnp.float32)

    ts = 8
    while h % ts:
        ts //= 2
    t_steps = h // ts
    lpt = ts * w // 128

    cost = pl.CostEstimate(
        flops=2 * n * h * w * (6 * c) * (4 * c),
        transcendentals=0,
        bytes_accessed=(n * h * w * c) * 4
        + (2 * (6 * c) * (2 * c)) * 2 + (n * h * 2 * w * 2 * c) * 4,
    )
    out3 = pl.pallas_call(
        _make_fused_kernel(h, w, c, ts),
        out_shape=jax.ShapeDtypeStruct((n, c, 4 * h * w), jnp.float32),
        grid=(n, t_steps),
        in_specs=[
            pl.BlockSpec((1, c, ts * w), lambda ni, ti: (ni, 0, ti)),
            pl.BlockSpec((1, c, 128),
                         lambda ni, ti: (ni, 0, jnp.maximum(ti * lpt - 1, 0))),
            pl.BlockSpec((1, c, 128),
                         lambda ni, ti: (ni, 0,
                                         jnp.minimum((ti + 1) * lpt,
                                                     h * w // 128 - 1))),
            pl.BlockSpec((2, 6 * c, 2 * c), lambda ni, ti: (0, 0, 0)),
            pl.BlockSpec((2, 1, 2 * c), lambda ni, ti: (0, 0, 0)),
        ],
        out_specs=pl.BlockSpec((1, c, 4 * ts * w), lambda ni, ti: (ni, 0, ti)),
        scratch_shapes=[pltpu.VMEM((ts + 2, w + 2, c), jnp.bfloat16)],
        compiler_params=pltpu.CompilerParams(
            dimension_semantics=("parallel", "arbitrary")),
        cost_estimate=cost,
    )(x3, x3, x3, w6, b2)

    return out3.reshape(n, c, 2 * h, 2 * w)


# speedup vs baseline: 27.8476x; 27.8476x over previous
import jax
import jax.numpy as jnp
from jax.experimental import pallas as pl
from jax.experimental.pallas import tpu as pltpu


def _make_fused_kernel(H, W, C, TS):
    TWO_C = 2 * C
    THREE_C = 3 * C
    SIX_C = 6 * C
    T = H // TS

    def _body(x_ref, w_ref, b_ref, o_ref, xp_ref, xt_ref):
        xt_ref[...] = jnp.transpose(x_ref[0].astype(jnp.bfloat16))

        zrow = jnp.zeros((1, W, C), jnp.bfloat16)
        zcol = jnp.zeros((TS + 2, 1, C), jnp.bfloat16)

        for t in range(T):
            xp_ref[:, 0:1, :] = zcol
            xp_ref[:, W + 1:W + 2, :] = zcol
            xp_ref[1:TS + 1, 1:W + 1, :] = (
                xt_ref[t * TS * W:(t * TS + TS) * W, :].reshape(TS, W, C))
            if t == 0:
                xp_ref[0:1, 1:W + 1, :] = zrow
            else:
                xp_ref[0:1, 1:W + 1, :] = (
                    xt_ref[(t * TS - 1) * W:t * TS * W, :].reshape(1, W, C))
            if t == T - 1:
                xp_ref[TS + 1:TS + 2, 1:W + 1, :] = zrow
            else:
                xp_ref[TS + 1:TS + 2, 1:W + 1, :] = (
                    xt_ref[(t * TS + TS) * W:(t * TS + TS + 1) * W, :]
                    .reshape(1, W, C))

            xp = xp_ref[...]
            taps = []
            for r in range(3):
                for s in range(3):
                    taps.append(xp[r:r + TS, s:s + W, :].reshape(TS * W, C))
            patches = jnp.concatenate(taps, axis=-1)

            acc0 = jnp.dot(patches[:, :SIX_C], w_ref[0],
                           preferred_element_type=jnp.float32) + b_ref[0]
            acc1 = jnp.dot(patches[:, THREE_C:], w_ref[1],
                           preferred_element_type=jnp.float32) + b_ref[1]

            o_ref[0:1, t * TS:(t + 1) * TS, 0:1, :, :] = (
                acc0.astype(o_ref.dtype).reshape(1, TS, 1, W, TWO_C))
            o_ref[0:1, t * TS:(t + 1) * TS, 1:2, :, :] = (
                acc1.astype(o_ref.dtype).reshape(1, TS, 1, W, TWO_C))

    return _body


def kernel(x, w_cat, b_cat):
    n, c, h, w = x.shape
    x3 = x.reshape(n, c, h * w)

    wr = w_cat.reshape(2, 3, 3 * c, 2 * c)
    w6 = jnp.stack([wr[0, 0:2].reshape(6 * c, 2 * c),
                    wr[1, 1:3].reshape(6 * c, 2 * c)]).astype(jnp.bfloat16)
    b2 = b_cat.astype(jnp.float32)

    ts = 16
    while h % ts:
        ts //= 2

    cost = pl.CostEstimate(
        flops=2 * n * h * w * (6 * c) * (4 * c),
        transcendentals=0,
        bytes_accessed=(n * h * w * c) * 4
        + (2 * (6 * c) * (2 * c) + n * h * 2 * w * 2 * c) * 2,
    )
    out6 = pl.pallas_call(
        _make_fused_kernel(h, w, c, ts),
        out_shape=jax.ShapeDtypeStruct((n, h, 2, w, 2 * c), jnp.bfloat16),
        grid=(n,),
        in_specs=[
            pl.BlockSpec((1, c, h * w), lambda ni: (ni, 0, 0)),
            pl.BlockSpec((2, 6 * c, 2 * c), lambda ni: (0, 0, 0)),
            pl.BlockSpec((2, 1, 2 * c), lambda ni: (0, 0, 0)),
        ],
        out_specs=pl.BlockSpec((1, h, 2, w, 2 * c),
                               lambda ni: (ni, 0, 0, 0, 0)),
        scratch_shapes=[pltpu.VMEM((ts + 2, w + 2, c), jnp.bfloat16),
                        pltpu.VMEM((h * w, c), jnp.bfloat16)],
        compiler_params=pltpu.CompilerParams(
            dimension_semantics=("parallel",)),
        cost_estimate=cost,
    )(x3, w6, b2)

    out_nhwc = out6.reshape(n, 2 * h, 2 * w, c)
    return jnp.transpose(out_nhwc, (0, 3, 1, 2)).astype(jnp.float32)
